# Initial kernel scaffold; baseline (speedup 1.0000x reference)
import jax
import jax.numpy as jnp
from jax.experimental import pallas as pl


def kernel(word_probs, active_likelihoods, active_sequences):
    raise NotImplementedError("write your pallas kernel here")


# speedup vs baseline: 4.6620x; 4.6620x over previous
import dataclasses
import functools

import jax
import jax.numpy as jnp
from jax import lax
from jax.experimental import pallas as pl
from jax.experimental.pallas import tpu as pltpu
from jax.experimental.pallas import tpu_sc as plsc

B = 32
BEAM = 8
V = 100000
SEQ_T = 16
OUT_T = SEQ_T + 1
FLAT = BEAM * V
L = 16
CH = 50000
NCH = FLAT // CH
CPB = V // CH
VPG = 25
GROUPS = CH // (VPG * L)
GPS = 25
NSUP = GROUPS // GPS
NSUB = 10
SUB = CH // NSUB
SEQ_OUT_W = 144

_NEG_INF = float("-inf")


def _iota():
    return lax.iota(jnp.int32, L)


def _bcast_lane_f32(vec, lane):
    picked = jnp.max(jnp.where(_iota() == lane, vec, _NEG_INF))
    return lax.broadcast_in_dim(picked, (L,), ())


def _bcast_lane_i32(vec, lane):
    picked = jnp.max(jnp.where(_iota() == lane, vec, -1))
    return lax.broadcast_in_dim(picked, (L,), ())


def _merge_topk(vals, idxs, likb, t_ref, i_ref, thr_ref, tadj_ref):
    xa, ia = plsc.sort_key_val(vals, idxs, descending=False)
    t_cur = t_ref[...]
    i_cur = i_ref[...]
    keep = t_cur >= xa
    m = jnp.where(keep, t_cur, xa)
    mi = jnp.where(keep, i_cur, ia)
    t_new, i_new = plsc.sort_key_val(m, mi, descending=True)
    t_ref[...] = t_new
    i_ref[...] = i_new
    thr = lax.broadcast_in_dim(jnp.min(t_new), (L,), ())
    thr_ref[...] = thr
    tadj_ref[...] = thr - likb


def _sc_body(wp_hbm, lik_hbm, seq_hbm,
             seq_out_hbm, cand_lik_hbm, compl_hbm, compl_lik_hbm, act_lik_hbm,
             buf_a, buf_b, gm_ref, sm_ref, lik_v, seqrow_v, outseq_v,
             t_ref, i_ref, thr_ref, tadj_ref,
             of_a, of_b, of_c, oi_a,
             sem_a, sem_b):
    nc = 2
    b = lax.axis_index("s") * nc + lax.axis_index("c")

    pltpu.sync_copy(lik_hbm.at[b], lik_v)
    lik16 = lik_v[...]

    t_ref[...] = jnp.full((L,), -jnp.inf, jnp.float32)
    i_ref[...] = jnp.zeros((L,), jnp.int32)
    thr_ref[...] = jnp.full((L,), -jnp.inf, jnp.float32)

    def _start_chunk(beam_i, off, buf, sem):
        for i in range(NSUB):
            pltpu.async_copy(wp_hbm.at[b, beam_i, pl.ds(off + i * SUB, SUB)],
                             buf.at[pl.ds(i * SUB, SUB)], sem)

    def _wait_chunk(beam_i, off, buf, sem):
        for i in range(NSUB):
            pltpu.make_async_copy(
                wp_hbm.at[b, beam_i, pl.ds(off + i * SUB, SUB)],
                buf.at[pl.ds(i * SUB, SUB)], sem).wait()

    _start_chunk(0, 0, buf_a, sem_a)

    def _tree_max(vs):
        tree = list(vs)
        while len(tree) > 1:
            lvl = [jnp.maximum(p, q) for p, q in zip(tree[::2], tree[1::2])]
            if len(tree) % 2:
                lvl.append(tree[-1])
            tree = lvl
        return tree[0]

    def _process_chunk(cc, j, buf, sem, nxt_buf, nxt_sem):
        beam = cc // CPB
        likb = _bcast_lane_f32(lik16, beam)
        tadj_ref[...] = thr_ref[...] - likb

        @pl.when(cc + 1 < NCH)
        def _():
            _start_chunk(beam + j, (1 - j) * CH, nxt_buf, nxt_sem)

        _wait_chunk(beam, j * CH, buf, sem)

        base_idx = cc * CH

        @plsc.parallel_loop(0, GROUPS, unroll=5)
        def _(g):
            base = g * (VPG * L)
            gm_ref[pl.ds(g * L, L)] = _tree_max(
                [buf[pl.ds(base + k * L, L)] for k in range(VPG)])

        for s in range(NSUP):
            sm_ref[pl.ds(s * L, L)] = _tree_max(
                [gm_ref[pl.ds((s * GPS + i) * L, L)] for i in range(GPS)])

        @pl.loop(0, NSUP)
        def _(s):
            sv = sm_ref[pl.ds(s * L, L)]

            @pl.when(jnp.any(sv > tadj_ref[...]))
            def _():
                @pl.loop(0, GPS)
                def _(gi):
                    g = s * GPS + gi
                    gv = gm_ref[pl.ds(g * L, L)]

                    @pl.when(jnp.any(gv > tadj_ref[...]))
                    def _():
                        gbase = g * (VPG * L)

                        @pl.loop(0, VPG)
                        def _(k):
                            x = buf[pl.ds(gbase + k * L, L)]

                            @pl.when(jnp.any(x > tadj_ref[...]))
                            def _():
                                vals = x + likb
                                idxs = (base_idx + gbase + k * L) + _iota()
                                _merge_topk(vals, idxs, likb,
                                            t_ref, i_ref, thr_ref, tadj_ref)

    @pl.loop(0, NCH, step=2)
    def _(c0):
        _process_chunk(c0, 0, buf_a, sem_a, buf_b, sem_b)
        _process_chunk(c0 + 1, 1, buf_b, sem_b, buf_a, sem_a)

    top = t_ref[...]
    top_i = i_ref[...]
    seq_i = top_i // V
    word_i = top_i % V

    pltpu.sync_copy(seq_hbm.at[b], seqrow_v)

    iot = _iota()
    for j in range(BEAM):
        sj = _bcast_lane_i32(seq_i, j)
        row = plsc.load_gather(seqrow_v, [sj * SEQ_T + iot])
        plsc.store_scatter(outseq_v, [iot + j * OUT_T], row)
    lane8 = iot < BEAM
    plsc.store_scatter(outseq_v, [jnp.minimum(iot, BEAM - 1) * OUT_T + SEQ_T],
                       word_i, mask=lane8)
    pltpu.sync_copy(outseq_v, seq_out_hbm.at[b])

    of_a[...] = top
    pltpu.sync_copy(of_a, cand_lik_hbm.at[b])
    compl = word_i == 1
    oi_a[...] = jnp.where(compl, 1, 0).astype(jnp.int32)
    pltpu.sync_copy(oi_a, compl_hbm.at[b])
    of_b[...] = jnp.where(compl, top / float(OUT_T), _NEG_INF)
    pltpu.sync_copy(of_b, compl_lik_hbm.at[b])
    of_c[...] = jnp.where(compl, _NEG_INF, top)
    pltpu.sync_copy(of_c, act_lik_hbm.at[b])


def _make_sc_call():
    mesh = plsc.VectorSubcoreMesh(core_axis_name="c", subcore_axis_name="s")
    cp = pltpu.CompilerParams()
    fields = pltpu.CompilerParams.__dataclass_fields__
    if "needs_layout_passes" in fields:
        cp = dataclasses.replace(cp, needs_layout_passes=False)
    if "use_tc_tiling_on_sc" in fields:
        cp = dataclasses.replace(cp, use_tc_tiling_on_sc=False)
    out_type = (
        jax.ShapeDtypeStruct((B, SEQ_OUT_W), jnp.int32),
        jax.ShapeDtypeStruct((B, L), jnp.float32),
        jax.ShapeDtypeStruct((B, L), jnp.int32),
        jax.ShapeDtypeStruct((B, L), jnp.float32),
        jax.ShapeDtypeStruct((B, L), jnp.float32),
    )
    scratch_types = [
        pltpu.VMEM((CH,), jnp.float32),
        pltpu.VMEM((CH,), jnp.float32),
        pltpu.VMEM((GROUPS * L,), jnp.float32),
        pltpu.VMEM((NSUP * L,), jnp.float32),
        pltpu.VMEM((L,), jnp.float32),
        pltpu.VMEM((BEAM * SEQ_T,), jnp.int32),
        pltpu.VMEM((SEQ_OUT_W,), jnp.int32),
        pltpu.VMEM((L,), jnp.float32),
        pltpu.VMEM((L,), jnp.int32),
        pltpu.VMEM((L,), jnp.float32),
        pltpu.VMEM((L,), jnp.float32),
        pltpu.VMEM((L,), jnp.float32),
        pltpu.VMEM((L,), jnp.float32),
        pltpu.VMEM((L,), jnp.float32),
        pltpu.VMEM((L,), jnp.int32),
        pltpu.SemaphoreType.DMA,
        pltpu.SemaphoreType.DMA,
    ]
    return pl.kernel(_sc_body, out_type=out_type, mesh=mesh,
                     scratch_types=scratch_types, compiler_params=cp)


_SC_CALL = _make_sc_call()


def kernel(word_probs, active_likelihoods, active_sequences):
    lik = jnp.concatenate(
        [active_likelihoods,
         jnp.zeros((B, L - BEAM), active_likelihoods.dtype)], axis=1)
    seq = active_sequences.reshape(B, BEAM * SEQ_T)
    seq_out, cand_lik, compl, compl_lik, act_lik = _SC_CALL(word_probs, lik, seq)
    candidate_sequences = seq_out[:, :BEAM * OUT_T].reshape(B, BEAM, OUT_T)
    return (candidate_sequences,
            cand_lik[:, :BEAM],
            compl[:, :BEAM].astype(bool),
            compl_lik[:, :BEAM],
            act_lik[:, :BEAM])


# speedup vs baseline: 7.4922x; 1.6071x over previous
import dataclasses
import functools

import jax
import jax.numpy as jnp
from jax import lax
from jax.experimental import pallas as pl
from jax.experimental.pallas import tpu as pltpu
from jax.experimental.pallas import tpu_sc as plsc

B = 32
BEAM = 8
V = 100000
SEQ_T = 16
OUT_T = SEQ_T + 1
L = 16
NT_FULL = V // 128
TCB = 48
W = TCB * 128
NCHT = 16
REM_T = NT_FULL - NCHT * TCB
REM_W = REM_T * 128
TAIL_OFF = NT_FULL * 128
TAIL_W = V - TAIL_OFF
NSUB = 8
SUBT = TCB // NSUB
STW = SUBT * 128
SGT = 8
SEQ_ROW_W = 256
ORW = 128

_NEG_INF = float("-inf")


def _iota():
    return lax.iota(jnp.int32, L)


def _bcast_lane_f32(vec, lane):
    picked = jnp.max(jnp.where(_iota() == lane, vec, _NEG_INF))
    return lax.broadcast_in_dim(picked, (L,), ())


def _bcast_lane_i32(vec, lane):
    picked = jnp.max(jnp.where(_iota() == lane, vec, -1))
    return lax.broadcast_in_dim(picked, (L,), ())


def _tree_max(vs):
    tree = list(vs)
    while len(tree) > 1:
        lvl = [jnp.maximum(p, q) for p, q in zip(tree[::2], tree[1::2])]
        if len(tree) % 2:
            lvl.append(tree[-1])
        tree = lvl
    return tree[0]


def _merge_topk(vals, idxs, t_ref, i_ref, thr_ref):
    xa, ia = plsc.sort_key_val(vals, idxs, descending=False)
    t_cur = t_ref[...]
    i_cur = i_ref[...]
    keep = t_cur >= xa
    m = jnp.where(keep, t_cur, xa)
    mi = jnp.where(keep, i_cur, ia)
    t_new, i_new = plsc.sort_key_val(m, mi, descending=True)
    t_ref[...] = t_new
    i_ref[...] = i_new
    thr_ref[...] = jnp.maximum(
        thr_ref[...], lax.broadcast_in_dim(jnp.min(t_new), (L,), ()))


def _sc_body(wp_hbm, tail_hbm, lik_hbm, seq_hbm,
             seq_out_hbm, cand_lik_hbm, compl_hbm, compl_lik_hbm, act_lik_hbm,
             buf_a, buf_b, gm_ref, sm_ref, lik_v, seqrow_v, outseq_v,
             t_ref, i_ref, thr_ref, wt_ref,
             of_a, of_b, of_c, oi_a,
             sem_a, sem_b):
    nc = 2
    b = lax.axis_index("s") * nc + lax.axis_index("c")

    pltpu.sync_copy(lik_hbm.at[pl.ds(b * ORW, L)], lik_v)
    lik16 = lik_v[...]
    likbs = [_bcast_lane_f32(lik16, r) for r in range(BEAM)]

    t_ref[...] = jnp.full((L,), -jnp.inf, jnp.float32)
    i_ref[...] = jnp.zeros((L,), jnp.int32)
    thr_ref[...] = jnp.full((L,), -jnp.inf, jnp.float32)
    wt_ref[...] = jnp.full((L,), -jnp.inf, jnp.float32)

    def _start_chunk(col0, ntiles, buf, sem):
        nsub = max(1, ntiles // SUBT)
        step = ntiles // nsub
        for i in range(nsub):
            w0 = i * step * 128
            w1 = (ntiles * 128 if i == nsub - 1 else (i + 1) * step * 128)
            pltpu.async_copy(wp_hbm.at[b, :, pl.ds(col0 + w0, w1 - w0)],
                             buf.at[:, pl.ds(w0, w1 - w0)], sem)

    def _wait_chunk(col0, ntiles, buf, sem):
        nsub = max(1, ntiles // SUBT)
        step = ntiles // nsub
        for i in range(nsub):
            w0 = i * step * 128
            w1 = (ntiles * 128 if i == nsub - 1 else (i + 1) * step * 128)
            pltpu.make_async_copy(wp_hbm.at[b, :, pl.ds(col0 + w0, w1 - w0)],
                                  buf.at[:, pl.ds(w0, w1 - w0)], sem).wait()

    def _phase_a(buf, ntiles, unroll=2):
        @plsc.parallel_loop(0, ntiles, unroll=unroll)
        def _(t):
            rows = []
            for r in range(BEAM):
                rm = _tree_max([buf[r, pl.ds(t * 128 + k * L, L)]
                                for k in range(8)])
                rows.append(rm + likbs[r])
            gm_ref[pl.ds(t * L, L)] = _tree_max(rows)

        nsup = (ntiles + SGT - 1) // SGT
        for s in range(nsup):
            t0, t1 = s * SGT, min((s + 1) * SGT, ntiles)
            sm_ref[pl.ds(s * L, L)] = _tree_max(
                [gm_ref[pl.ds(t * L, L)] for t in range(t0, t1)])
        return nsup

    def _phase_b(buf, ntiles, nsup, base_col):
        @pl.loop(0, nsup)
        def _(s):
            sv = sm_ref[pl.ds(s * L, L)]

            @pl.when(jnp.any(sv > thr_ref[...]))
            def _():
                @pl.loop(0, SGT)
                def _(i):
                    ti = s * SGT + i

                    @pl.when(ti < ntiles)
                    def _():
                        gv = gm_ref[pl.ds(ti * L, L)]

                        @pl.when(jnp.any(gv > thr_ref[...]))
                        def _():
                            for r in range(BEAM):
                                tadj = thr_ref[...] - likbs[r]

                                @pl.loop(0, 8)
                                def _(k):
                                    x = buf[r, pl.ds(ti * 128 + k * L, L)]

                                    @pl.when(jnp.any(x > tadj))
                                    def _():
                                        idxs = (r * V + base_col + ti * 128
                                                + k * L) + _iota()
                                        _merge_topk(x + likbs[r], idxs,
                                                    t_ref, i_ref, thr_ref)

    _start_chunk(0, TCB, buf_a, sem_a)

    def _process_chunk(cc, j, buf, sem, nxt_buf, nxt_sem):
        @pl.when(cc + 1 < NCHT)
        def _():
            _start_chunk((cc + 1) * W, TCB, nxt_buf, nxt_sem)

        @pl.when(cc + 1 == NCHT)
        def _():
            _start_chunk(NCHT * W, REM_T, nxt_buf, nxt_sem)

        _wait_chunk(cc * W, TCB, buf, sem)
        nsup = _phase_a(buf, TCB)

        @pl.when(cc == 0)
        def _():
            @pl.loop(0, TCB)
            def _(t):
                gv = gm_ref[pl.ds(t * L, L)]

                @pl.when(jnp.any(gv > thr_ref[...]))
                def _():
                    ga, _u = plsc.sort_key_val(gv, _iota(), descending=False)
                    wcur = wt_ref[...]
                    wm = jnp.where(wcur >= ga, wcur, ga)
                    ws, _u2 = plsc.sort_key_val(wm, _iota(), descending=True)
                    wt_ref[...] = ws
                    thr_ref[...] = jnp.maximum(
                        thr_ref[...],
                        lax.broadcast_in_dim(jnp.min(ws), (L,), ()))

        _phase_b(buf, TCB, nsup, cc * W)

    @pl.loop(0, NCHT, step=2)
    def _(c0):
        _process_chunk(c0, 0, buf_a, sem_a, buf_b, sem_b)
        _process_chunk(c0 + 1, 1, buf_b, sem_b, buf_a, sem_a)

    ntail = BEAM * TAIL_W
    pltpu.async_copy(tail_hbm.at[pl.ds(b * ntail, ntail)],
                     buf_b.at[0, pl.ds(0, ntail)], sem_b)
    _wait_chunk(NCHT * W, REM_T, buf_a, sem_a)
    nsup = _phase_a(buf_a, REM_T, unroll=1)
    _phase_b(buf_a, REM_T, nsup, NCHT * W)

    pltpu.make_async_copy(tail_hbm.at[pl.ds(b * ntail, ntail)],
                          buf_b.at[0, pl.ds(0, ntail)], sem_b).wait()
    for jv in range(ntail // L):
        r = jv // (TAIL_W // L)
        tadj = thr_ref[...] - likbs[r]
        x = buf_b[0, pl.ds(jv * L, L)]

        @pl.when(jnp.any(x > tadj))
        def _():
            idxs = (r * V + TAIL_OFF + (jv % (TAIL_W // L)) * L) + _iota()
            _merge_topk(x + likbs[r], idxs, t_ref, i_ref, thr_ref)

    top = t_ref[...]
    top_i = i_ref[...]
    seq_i = top_i // V
    word_i = top_i % V

    pltpu.sync_copy(seq_hbm.at[pl.ds(b * (BEAM * SEQ_T), BEAM * SEQ_T)],
                    seqrow_v)

    iot = _iota()
    for j in range(BEAM):
        sj = _bcast_lane_i32(seq_i, j)
        row = plsc.load_gather(seqrow_v, [sj * SEQ_T + iot])
        plsc.store_scatter(outseq_v, [iot + j * OUT_T], row)
    lane8 = iot < BEAM
    plsc.store_scatter(outseq_v, [jnp.minimum(iot, BEAM - 1) * OUT_T + SEQ_T],
                       word_i, mask=lane8)
    pltpu.sync_copy(outseq_v,
                    seq_out_hbm.at[pl.ds(b * SEQ_ROW_W, BEAM * OUT_T)])

    of_a[...] = top
    pltpu.sync_copy(of_a, cand_lik_hbm.at[pl.ds(b * ORW, L)])
    compl = word_i == 1
    oi_a[...] = jnp.where(compl, 1, 0).astype(jnp.int32)
    pltpu.sync_copy(oi_a, compl_hbm.at[pl.ds(b * ORW, L)])
    of_b[...] = jnp.where(compl, top / float(OUT_T), _NEG_INF)
    pltpu.sync_copy(of_b, compl_lik_hbm.at[pl.ds(b * ORW, L)])
    of_c[...] = jnp.where(compl, _NEG_INF, top)
    pltpu.sync_copy(of_c, act_lik_hbm.at[pl.ds(b * ORW, L)])


def _make_sc_call():
    mesh = plsc.VectorSubcoreMesh(core_axis_name="c", subcore_axis_name="s")
    cp = pltpu.CompilerParams()
    fields = pltpu.CompilerParams.__dataclass_fields__
    if "needs_layout_passes" in fields:
        cp = dataclasses.replace(cp, needs_layout_passes=False)
    if "use_tc_tiling_on_sc" in fields:
        cp = dataclasses.replace(cp, use_tc_tiling_on_sc=True)
    out_type = (
        jax.ShapeDtypeStruct((B * SEQ_ROW_W,), jnp.int32),
        jax.ShapeDtypeStruct((B * ORW,), jnp.float32),
        jax.ShapeDtypeStruct((B * ORW,), jnp.int32),
        jax.ShapeDtypeStruct((B * ORW,), jnp.float32),
        jax.ShapeDtypeStruct((B * ORW,), jnp.float32),
    )
    scratch_types = [
        pltpu.VMEM((BEAM, W), jnp.float32),
        pltpu.VMEM((BEAM, W), jnp.float32),
        pltpu.VMEM((TCB * L,), jnp.float32),
        pltpu.VMEM(((TCB // SGT + 1) * L,), jnp.float32),
        pltpu.VMEM((L,), jnp.float32),
        pltpu.VMEM((BEAM * SEQ_T,), jnp.int32),
        pltpu.VMEM((BEAM * OUT_T,), jnp.int32),
        pltpu.VMEM((L,), jnp.float32),
        pltpu.VMEM((L,), jnp.int32),
        pltpu.VMEM((L,), jnp.float32),
        pltpu.VMEM((L,), jnp.float32),
        pltpu.VMEM((L,), jnp.float32),
        pltpu.VMEM((L,), jnp.float32),
        pltpu.VMEM((L,), jnp.float32),
        pltpu.VMEM((L,), jnp.int32),
        pltpu.SemaphoreType.DMA,
        pltpu.SemaphoreType.DMA,
    ]
    return pl.kernel(_sc_body, out_type=out_type, mesh=mesh,
                     scratch_types=scratch_types, compiler_params=cp)


_SC_CALL = _make_sc_call()


def kernel(word_probs, active_likelihoods, active_sequences):
    tail = word_probs[:, :, TAIL_OFF:].reshape(-1)
    lik = jnp.pad(active_likelihoods, ((0, 0), (0, ORW - BEAM))).reshape(-1)
    seq = active_sequences.reshape(-1)
    seq_out, cand_lik, compl, compl_lik, act_lik = _SC_CALL(
        word_probs, tail, lik, seq)
    candidate_sequences = (
        seq_out.reshape(B, SEQ_ROW_W)[:, :BEAM * OUT_T].reshape(B, BEAM, OUT_T))
    return (candidate_sequences,
            cand_lik.reshape(B, ORW)[:, :BEAM],
            compl.reshape(B, ORW)[:, :BEAM].astype(bool),
            compl_lik.reshape(B, ORW)[:, :BEAM],
            act_lik.reshape(B, ORW)[:, :BEAM])
